# baseline (speedup 1.0000x reference)
import functools

import jax
import jax.numpy as jnp
from jax.experimental import pallas as pl
from jax.experimental.pallas import tpu as pltpu

_RC2 = 36.0
_EPS = 1e-12


def _body(d_pl_ref, pos_ref, vrow_ref, vcol_ref,
          vec_ref, sod_ref, mask_ref, *, SC):
    p = pos_ref[0]
    vrowb = vrow_ref[0] > 0.0
    vcolb = vcol_ref[0] > 0.0
    for k in range(SC):
        dp = d_pl_ref[0, k]
        vx = dp[0:1, :] - p[:, 0:1]
        vy = dp[1:2, :] - p[:, 1:2]
        vz = dp[2:3, :] - p[:, 2:3]
        sod = vx * vx + vy * vy + vz * vz
        m = (sod < _RC2) & (sod > _EPS) & vrowb & vcolb
        sod_ref[0, k] = jnp.where(m, sod, 0.0)
        mask_ref[0, k] = m
        vec_ref[0, k, 0] = jnp.where(m, vx, 0.0)
        vec_ref[0, k, 1] = jnp.where(m, vy, 0.0)
        vec_ref[0, k, 2] = jnp.where(m, vz, 0.0)


@jax.jit
def kernel(pos, cel, sft_cel, ent):
    B, N, _ = pos.shape
    S = sft_cel.shape[0]
    f32 = pos.dtype

    sft_xyz = jnp.einsum('sk,bkl->bsl', sft_cel.astype(f32), cel)
    d = pos[:, None, :, :] + sft_xyz[:, :, None, :]
    d_pl = d.transpose(0, 1, 3, 2)
    validf = (ent > 0).astype(f32)
    vrow = validf.reshape(B, 1, N)
    vcol = validf.reshape(B, N, 1)

    SC = 3
    grid = (B, S // SC)
    vec_out, sod_out, mask_out = pl.pallas_call(
        functools.partial(_body, SC=SC),
        grid=grid,
        in_specs=[
            pl.BlockSpec((1, SC, 3, N), lambda b, s: (b, s, 0, 0)),
            pl.BlockSpec((1, N, 3), lambda b, s: (b, 0, 0)),
            pl.BlockSpec((1, 1, N), lambda b, s: (b, 0, 0)),
            pl.BlockSpec((1, N, 1), lambda b, s: (b, 0, 0)),
        ],
        out_specs=[
            pl.BlockSpec((1, SC, 3, N, N), lambda b, s: (b, s, 0, 0, 0)),
            pl.BlockSpec((1, SC, N, N), lambda b, s: (b, s, 0, 0)),
            pl.BlockSpec((1, SC, N, N), lambda b, s: (b, s, 0, 0)),
        ],
        out_shape=[
            jax.ShapeDtypeStruct((B, S, 3, N, N), f32),
            jax.ShapeDtypeStruct((B, S, N, N), f32),
            jax.ShapeDtypeStruct((B, S, N, N), jnp.bool_),
        ],
    )(d_pl, pos, vrow, vcol)

    return vec_out.transpose(0, 1, 3, 4, 2), sod_out, mask_out
